# Initial kernel scaffold; baseline (speedup 1.0000x reference)
import jax
import jax.numpy as jnp
from jax.experimental import pallas as pl


def kernel(input, weight):
    raise NotImplementedError("write your pallas kernel here")


# speedup vs baseline: 1.4837x; 1.4837x over previous
import jax
import jax.numpy as jnp
from jax import lax
from jax.experimental import pallas as pl
from jax.experimental.pallas import tpu as pltpu
from jax.experimental.pallas import tpu_sc as plsc

B = 16384
H = 50
D = 64
ROWL = 104
PAIRS = B // 2
NC, NS = 2, 16
NW = NC * NS
PPW = PAIRS // NW
K = 4
CHUNKS = PPW // K
BPW = B // NW
NV = D // 16


def _body(weight_hbm, idx_hbm, out_hbm, idx_v, rows_v, out_v, sem):
    wid = lax.axis_index("s") * NC + lax.axis_index("c")
    pair_base = wid * PPW
    zero = jnp.zeros((16,), jnp.float32)

    def chunk(ci, carry):
        pltpu.sync_copy(idx_hbm.at[pl.ds(pair_base + ci * K, K)], idx_v)
        descs = [
            pltpu.async_copy(weight_hbm.at[idx_v.at[j]], rows_v.at[j], sem)
            for j in range(K)
        ]
        for d in descs:
            d.wait()
        for j in range(K):
            for r in range(2):
                def red(l, acc, _j=j, _r=r):
                    return tuple(
                        acc[v] + rows_v[_j, _r * H + l, pl.ds(v * 16, 16)]
                        for v in range(NV)
                    )
                acc = lax.fori_loop(0, H, red, (zero,) * NV)
                orow = ci * (2 * K) + 2 * j + r
                for v in range(NV):
                    out_v[orow, pl.ds(v * 16, 16)] = acc[v] * (1.0 / H)
        return carry

    lax.fori_loop(0, CHUNKS, chunk, 0)
    pltpu.sync_copy(out_v, out_hbm.at[pl.ds(wid * BPW, BPW)])


_sc_call = pl.kernel(
    _body,
    out_type=jax.ShapeDtypeStruct((B, D), jnp.float32),
    mesh=plsc.VectorSubcoreMesh(
        core_axis_name="c", subcore_axis_name="s", num_cores=NC, num_subcores=NS
    ),
    scratch_types=[
        pltpu.VMEM((K, ROWL), jnp.int32),
        pltpu.VMEM((K, ROWL, D), jnp.float32),
        pltpu.VMEM((BPW, D), jnp.float32),
        pltpu.SemaphoreType.DMA,
    ],
    compiler_params=pltpu.CompilerParams(use_tc_tiling_on_sc=False),
)


def kernel(input, weight):
    idx = input.astype(jnp.int32).reshape(PAIRS, 2 * H)
    idx = jnp.pad(idx, ((0, 0), (0, ROWL - 2 * H)))
    return _sc_call(weight, idx)
